# Initial kernel scaffold; baseline (speedup 1.0000x reference)
import jax
import jax.numpy as jnp
from jax.experimental import pallas as pl


def kernel(x, edge_index, edge_type, weights, roots, biases):
    raise NotImplementedError("write your pallas kernel here")


# speedup vs baseline: 7.2956x; 7.2956x over previous
import functools

import jax
import jax.numpy as jnp
from jax import lax
from jax.experimental import pallas as pl
from jax.experimental.pallas import tpu as pltpu
from jax.experimental.pallas import tpu_sc as plsc

N = 10000
E = 320000
D = 128
R = 4
NLAYERS = 2

NC = 2
NS = 16
NW = NC * NS
EPW = E // NW
CH = 80
NCHUNK = EPW // CH
CPAD = 40960
CNT_PER_TILE = CPAD // NS
NPAD = 10240
ROWS_PER_TILE = NPAD // NS
ZROWS = 128

_MESH = plsc.VectorSubcoreMesh(
    core_axis_name="c", subcore_axis_name="s", num_cores=NC, num_subcores=NS)


@functools.partial(
    pl.kernel,
    out_type=(
        jax.ShapeDtypeStruct((E,), jnp.int32),
        jax.ShapeDtypeStruct((E,), jnp.int32),
        jax.ShapeDtypeStruct((NC, CPAD), jnp.float32),
    ),
    mesh=_MESH,
    scratch_types=[
        pltpu.VMEM((CH,), jnp.int32),
        pltpu.VMEM((CH,), jnp.int32),
        pltpu.VMEM((CH,), jnp.int32),
        pltpu.VMEM((CH,), jnp.int32),
        pltpu.VMEM((CH,), jnp.int32),
        pltpu.VMEM((CH,), jnp.float32),
        pltpu.VMEM((CNT_PER_TILE,), jnp.float32),
        pltpu.VMEM_SHARED((CPAD,), jnp.float32),
    ],
)
def _sc_counts(src_hbm, dst_hbm, typ_hbm, gout_hbm, cout_hbm, cnt_hbm,
               sv, dv, tv, gv, cv, ones, zb, cnt_sh):
    c = lax.axis_index("c")
    s = lax.axis_index("s")
    wid = c * NS + s

    zero16 = jnp.zeros((16,), jnp.float32)
    one16 = jnp.ones((16,), jnp.float32)

    def _zb_body(i, carry):
        zb[pl.ds(i * 16, 16)] = zero16
        return carry
    lax.fori_loop(0, CNT_PER_TILE // 16, _zb_body, 0)
    for j in range(CH // 16):
        ones[pl.ds(j * 16, 16)] = one16

    pltpu.sync_copy(zb, cnt_sh.at[pl.ds(s * CNT_PER_TILE, CNT_PER_TILE)])
    plsc.subcore_barrier()

    def _chunk(i, carry):
        base = wid * EPW + i * CH
        pltpu.sync_copy(src_hbm.at[pl.ds(base, CH)], sv)
        pltpu.sync_copy(dst_hbm.at[pl.ds(base, CH)], dv)
        pltpu.sync_copy(typ_hbm.at[pl.ds(base, CH)], tv)
        for j in range(CH // 16):
            sl = pl.ds(j * 16, 16)
            t_n = tv[sl] * N
            gv[sl] = t_n + sv[sl]
            cv[sl] = t_n + dv[sl]
        pltpu.sync_copy(gv, gout_hbm.at[pl.ds(base, CH)])
        pltpu.sync_copy(cv, cout_hbm.at[pl.ds(base, CH)])
        pltpu.sync_copy(ones, cnt_sh.at[cv], add=True)
        return carry
    lax.fori_loop(0, NCHUNK, _chunk, 0)

    plsc.subcore_barrier()
    sl = pl.ds(s * CNT_PER_TILE, CNT_PER_TILE)
    pltpu.sync_copy(cnt_sh.at[sl], cnt_hbm.at[c, sl])


@functools.partial(
    pl.kernel,
    out_type=jax.ShapeDtypeStruct((NC, NPAD, D), jnp.float32),
    mesh=_MESH,
    scratch_types=[
        pltpu.VMEM((CH,), jnp.int32),
        pltpu.VMEM((CH,), jnp.int32),
        pltpu.VMEM((CH,), jnp.int32),
        pltpu.VMEM((CH,), jnp.float32),
        pltpu.VMEM((CH, D), jnp.float32),
        pltpu.VMEM((ZROWS, D), jnp.float32),
        pltpu.VMEM_SHARED((NPAD, D), jnp.float32),
        pltpu.SemaphoreType.DMA,
        pltpu.SemaphoreType.DMA,
    ],
)
def _sc_aggregate(y_hbm, gidx_hbm, dst_hbm, cidx_hbm, inv_hbm, part_hbm,
                  gv, dv, cv, sval, rows, zb, acc, sem_a, sem_b):
    c = lax.axis_index("c")
    s = lax.axis_index("s")
    wid = c * NS + s

    zero16 = jnp.zeros((16,), jnp.float32)

    def _zb_body(i, carry):
        r = i // (D // 16)
        j = lax.rem(i, D // 16)
        zb[r, pl.ds(j * 16, 16)] = zero16
        return carry
    lax.fori_loop(0, ZROWS * (D // 16), _zb_body, 0)

    for k in range(ROWS_PER_TILE // ZROWS):
        pltpu.sync_copy(zb, acc.at[pl.ds(s * ROWS_PER_TILE + k * ZROWS, ZROWS)])
    plsc.subcore_barrier()

    def _chunk(i, carry):
        base = wid * EPW + i * CH
        pltpu.sync_copy(gidx_hbm.at[pl.ds(base, CH)], gv)
        pltpu.sync_copy(dst_hbm.at[pl.ds(base, CH)], dv)
        pltpu.sync_copy(cidx_hbm.at[pl.ds(base, CH)], cv)
        pltpu.async_copy(inv_hbm.at[cv], sval, sem_a).wait()
        pltpu.async_copy(y_hbm.at[gv], rows, sem_b).wait()

        def _scale(g, icarry):
            svec = sval[pl.ds(g * 16, 16)]
            for k in range(16):
                sc = svec[k]
                e = g * 16 + k
                for j in range(D // 16):
                    sl = pl.ds(j * 16, 16)
                    rows[e, sl] = rows[e, sl] * sc
            return icarry
        lax.fori_loop(0, CH // 16, _scale, 0)

        pltpu.sync_copy(rows, acc.at[dv], add=True)
        return carry
    lax.fori_loop(0, NCHUNK, _chunk, 0)

    plsc.subcore_barrier()
    for k in range(ROWS_PER_TILE // ZROWS):
        sl = pl.ds(s * ROWS_PER_TILE + k * ZROWS, ZROWS)
        pltpu.sync_copy(acc.at[sl], part_hbm.at[c, sl])


def _inv_body(cnt_ref, inv_ref):
    total = cnt_ref[0] + cnt_ref[1]
    inv_ref[...] = 1.0 / jnp.maximum(total, 1.0)


def _tc_inv_counts(cnt):
    cnt2 = cnt.reshape(NC, CPAD // 128, 128)
    inv = pl.pallas_call(
        _inv_body,
        out_shape=jax.ShapeDtypeStruct((CPAD // 128, 128), jnp.float32),
    )(cnt2)
    return inv.reshape(CPAD)


_BN = 1000


def _transform_body(h_ref, w_ref, root_ref, bias_ref, y_ref, base_ref):
    h = h_ref[...]
    base_ref[...] = jnp.dot(h, root_ref[...],
                            preferred_element_type=jnp.float32) + bias_ref[...]
    for r in range(R):
        y_ref[r] = jnp.dot(h, w_ref[r], preferred_element_type=jnp.float32)


def _tc_transform(h, w, root, bias):
    return pl.pallas_call(
        _transform_body,
        grid=(N // _BN,),
        in_specs=[
            pl.BlockSpec((_BN, D), lambda i: (i, 0)),
            pl.BlockSpec((R, D, D), lambda i: (0, 0, 0)),
            pl.BlockSpec((D, D), lambda i: (0, 0)),
            pl.BlockSpec((1, D), lambda i: (0, 0)),
        ],
        out_specs=[
            pl.BlockSpec((R, _BN, D), lambda i: (0, i, 0)),
            pl.BlockSpec((_BN, D), lambda i: (i, 0)),
        ],
        out_shape=[
            jax.ShapeDtypeStruct((R, N, D), jnp.float32),
            jax.ShapeDtypeStruct((N, D), jnp.float32),
        ],
    )(h, w, root, bias.reshape(1, D))


def _combine_body(base_ref, part_ref, out_ref):
    out_ref[...] = jnp.maximum(base_ref[...] + part_ref[0] + part_ref[1], 0.0)


def _tc_combine(base, parts):
    return pl.pallas_call(
        _combine_body,
        grid=(N // _BN,),
        in_specs=[
            pl.BlockSpec((_BN, D), lambda i: (i, 0)),
            pl.BlockSpec((NC, _BN, D), lambda i: (0, i, 0)),
        ],
        out_specs=pl.BlockSpec((_BN, D), lambda i: (i, 0)),
        out_shape=jax.ShapeDtypeStruct((N, D), jnp.float32),
    )(base, parts)


@jax.jit
def kernel(x, edge_index, edge_type, weights, roots, biases):
    src = edge_index[0].astype(jnp.int32)
    dst = edge_index[1].astype(jnp.int32)
    typ = edge_type.astype(jnp.int32)

    gidx, cidx, cnt = _sc_counts(src, dst, typ)
    inv = _tc_inv_counts(cnt)

    h = x
    for l in range(NLAYERS):
        y, base = _tc_transform(h, weights[l], roots[l], biases[l])
        parts = _sc_aggregate(y.reshape(R * N, D), gidx, dst, cidx, inv)
        h = _tc_combine(base, parts)
    return h
